# Initial kernel scaffold; baseline (speedup 1.0000x reference)
import jax
import jax.numpy as jnp
from jax import lax
from jax.experimental import pallas as pl
from jax.experimental.pallas import tpu as pltpu
from jax.experimental.pallas import tpu_sc as plsc

N = 10000
F = 128
E = 320000

NTILES = 32
CHUNK = 128
NP = 10112
ROWS_PER_TILE = NP // 16
EP = 323584
EDGES_PER_TILE = EP // NTILES
CHUNKS_PER_TILE = EDGES_PER_TILE // CHUNK

_mesh = plsc.VectorSubcoreMesh(core_axis_name="c", subcore_axis_name="s")


def _deg_body(dst_hbm, ones_hbm, zero_hbm, out_hbm, didx, ones_v, hist):
    c = lax.axis_index("c")
    s = lax.axis_index("s")
    wid = c * 16 + s
    r0 = s * ROWS_PER_TILE
    pltpu.sync_copy(zero_hbm.at[pl.ds(r0, ROWS_PER_TILE)],
                    hist.at[pl.ds(r0, ROWS_PER_TILE)])
    pltpu.sync_copy(ones_hbm, ones_v)
    plsc.subcore_barrier()
    base = wid * EDGES_PER_TILE

    @pl.loop(0, CHUNKS_PER_TILE)
    def _(k):
        pltpu.sync_copy(dst_hbm.at[pl.ds(base + k * CHUNK, CHUNK)], didx)
        pltpu.sync_copy(ones_v, hist.at[didx], add=True)

    plsc.subcore_barrier()
    pltpu.sync_copy(hist.at[pl.ds(r0, ROWS_PER_TILE)],
                    out_hbm.at[c, pl.ds(r0, ROWS_PER_TILE)])


_deg_call = pl.kernel(
    _deg_body,
    out_type=jax.ShapeDtypeStruct((2, NP, 16), jnp.float32),
    mesh=_mesh,
    scratch_types=[
        pltpu.VMEM((CHUNK,), jnp.int32),
        pltpu.VMEM((CHUNK, 16), jnp.float32),
        pltpu.VMEM_SHARED((NP, 16), jnp.float32),
    ],
)


def _conv_body(g_hbm, src_hbm, dst_hbm, zero_hbm, out_hbm,
               sidx, didx, rows, acc, sem):
    c = lax.axis_index("c")
    s = lax.axis_index("s")
    wid = c * 16 + s
    r0 = s * ROWS_PER_TILE
    pltpu.sync_copy(zero_hbm.at[pl.ds(r0, ROWS_PER_TILE)],
                    acc.at[pl.ds(r0, ROWS_PER_TILE)])
    plsc.subcore_barrier()
    base = wid * EDGES_PER_TILE

    @pl.loop(0, CHUNKS_PER_TILE)
    def _(k):
        pltpu.sync_copy(src_hbm.at[pl.ds(base + k * CHUNK, CHUNK)], sidx)
        pltpu.sync_copy(dst_hbm.at[pl.ds(base + k * CHUNK, CHUNK)], didx)
        pltpu.async_copy(g_hbm.at[sidx], rows, sem).wait()
        pltpu.sync_copy(rows, acc.at[didx], add=True)

    plsc.subcore_barrier()
    pltpu.sync_copy(acc.at[pl.ds(r0, ROWS_PER_TILE)],
                    out_hbm.at[c, pl.ds(r0, ROWS_PER_TILE)])


_conv_call = pl.kernel(
    _conv_body,
    out_type=jax.ShapeDtypeStruct((2, NP, F), jnp.float32),
    mesh=_mesh,
    scratch_types=[
        pltpu.VMEM((CHUNK,), jnp.int32),
        pltpu.VMEM((CHUNK,), jnp.int32),
        pltpu.VMEM((CHUNK, F), jnp.float32),
        pltpu.VMEM_SHARED((NP, F), jnp.float32),
        pltpu.SemaphoreType.DMA,
    ],
)


def _dis_from(degp_ref):
    deg = degp_ref[0, :, 0:1] + degp_ref[1, :, 0:1] + 1.0
    return lax.rsqrt(jnp.maximum(deg, 1e-12))


def _tc_in(x_ref, w_ref, b_ref, o_ref):
    o_ref[...] = (jnp.dot(x_ref[...], w_ref[...],
                          preferred_element_type=jnp.float32) + b_ref[...])


def _tc_pre(degp_ref, h_ref, w_ref, o_ref):
    dis = _dis_from(degp_ref)
    hw = jnp.dot(h_ref[...], w_ref[...], preferred_element_type=jnp.float32)
    o_ref[...] = hw * dis


def _tc_mid(degp_ref, p_ref, g_ref, w_ref, bprev_ref, o_ref):
    dis = _dis_from(degp_ref)
    acc = p_ref[0] + p_ref[1] + g_ref[...]
    h = jnp.maximum(acc * dis + bprev_ref[...], 0.0)
    hw = jnp.dot(h, w_ref[...], preferred_element_type=jnp.float32)
    o_ref[...] = hw * dis


def _tc_out(degp_ref, p_ref, g_ref, w_ref, bprev_ref, b_ref, o_ref):
    dis = _dis_from(degp_ref)
    acc = p_ref[0] + p_ref[1] + g_ref[...]
    h = jnp.maximum(acc * dis + bprev_ref[...], 0.0)
    out = jnp.dot(h, w_ref[...], preferred_element_type=jnp.float32) + b_ref[...]
    o_ref[...] = out[:N]


_f32 = jnp.float32
_tc_in_call = pl.pallas_call(
    _tc_in, out_shape=jax.ShapeDtypeStruct((NP, F), _f32))
_tc_pre_call = pl.pallas_call(
    _tc_pre, out_shape=jax.ShapeDtypeStruct((NP, F), _f32))
_tc_mid_call = pl.pallas_call(
    _tc_mid, out_shape=jax.ShapeDtypeStruct((NP, F), _f32))
_tc_out_call = pl.pallas_call(
    _tc_out, out_shape=jax.ShapeDtypeStruct((N, F), _f32))


def kernel(x, edge_index, W_in, b_in, W0, b0, W1, b1, W_out, b_out):
    src = edge_index[0]
    dst = edge_index[1]
    pad_e = EP - E
    src_p = jnp.concatenate([src, jnp.zeros((pad_e,), jnp.int32)])
    dst_p = jnp.concatenate([dst, jnp.full((pad_e,), N, jnp.int32)])
    x_p = jnp.pad(x, ((0, NP - N), (0, 0)))
    z16 = jnp.zeros((NP, 16), _f32)
    z128 = jnp.zeros((NP, F), _f32)
    ones16 = jnp.ones((CHUNK, 16), _f32)

    degp = _deg_call(dst_p, ones16, z16)
    h = _tc_in_call(x_p, W_in, b_in.reshape(1, F))
    g0 = _tc_pre_call(degp, h, W0)
    p0 = _conv_call(g0, src_p, dst_p, z128)
    g1 = _tc_mid_call(degp, p0, g0, W1, b0.reshape(1, F))
    p1 = _conv_call(g1, src_p, dst_p, z128)
    return _tc_out_call(degp, p1, g1, W_out,
                        b1.reshape(1, F), b_out.reshape(1, F))


# speedup vs baseline: 10.8600x; 10.8600x over previous
import jax
import jax.numpy as jnp
from jax import lax
from jax.experimental import pallas as pl
from jax.experimental.pallas import tpu as pltpu
from jax.experimental.pallas import tpu_sc as plsc

N = 10000
F = 128
E = 320000

NTILES = 32
CHUNK = 128
NP = 10112
ROWS_PER_TILE = NP // 16
EP = 323584
EDGES_PER_TILE = EP // NTILES
CHUNKS_PER_TILE = EDGES_PER_TILE // CHUNK

_mesh = plsc.VectorSubcoreMesh(core_axis_name="c", subcore_axis_name="s")


def _deg_body(dst_hbm, ones_hbm, zero_hbm, out_hbm, didx, ones_v, hist):
    c = lax.axis_index("c")
    s = lax.axis_index("s")
    wid = c * 16 + s
    r0 = s * ROWS_PER_TILE
    pltpu.sync_copy(zero_hbm.at[pl.ds(r0, ROWS_PER_TILE)],
                    hist.at[pl.ds(r0, ROWS_PER_TILE)])
    pltpu.sync_copy(ones_hbm, ones_v)
    plsc.subcore_barrier()
    base = wid * EDGES_PER_TILE

    @pl.loop(0, CHUNKS_PER_TILE)
    def _(k):
        pltpu.sync_copy(dst_hbm.at[pl.ds(base + k * CHUNK, CHUNK)], didx)
        pltpu.sync_copy(ones_v, hist.at[didx], add=True)

    plsc.subcore_barrier()
    pltpu.sync_copy(hist.at[pl.ds(r0, ROWS_PER_TILE)],
                    out_hbm.at[c, pl.ds(r0, ROWS_PER_TILE)])


_deg_call = pl.kernel(
    _deg_body,
    out_type=jax.ShapeDtypeStruct((2, NP, F), jnp.float32),
    mesh=_mesh,
    scratch_types=[
        pltpu.VMEM((CHUNK,), jnp.int32),
        pltpu.VMEM((CHUNK, F), jnp.float32),
        pltpu.VMEM_SHARED((NP, F), jnp.float32),
    ],
)


def _conv_body(g_hbm, src_hbm, dst_hbm, zero_hbm, out_hbm,
               sidx, didx, rows, acc, sem):
    c = lax.axis_index("c")
    s = lax.axis_index("s")
    wid = c * 16 + s
    r0 = s * ROWS_PER_TILE
    pltpu.sync_copy(zero_hbm.at[pl.ds(r0, ROWS_PER_TILE)],
                    acc.at[pl.ds(r0, ROWS_PER_TILE)])
    plsc.subcore_barrier()
    base = wid * EDGES_PER_TILE

    @pl.loop(0, CHUNKS_PER_TILE)
    def _(k):
        pltpu.sync_copy(src_hbm.at[pl.ds(base + k * CHUNK, CHUNK)], sidx)
        pltpu.sync_copy(dst_hbm.at[pl.ds(base + k * CHUNK, CHUNK)], didx)
        pltpu.async_copy(g_hbm.at[sidx], rows, sem).wait()
        pltpu.sync_copy(rows, acc.at[didx], add=True)

    plsc.subcore_barrier()
    pltpu.sync_copy(acc.at[pl.ds(r0, ROWS_PER_TILE)],
                    out_hbm.at[c, pl.ds(r0, ROWS_PER_TILE)])


_conv_call = pl.kernel(
    _conv_body,
    out_type=jax.ShapeDtypeStruct((2, NP, F), jnp.float32),
    mesh=_mesh,
    scratch_types=[
        pltpu.VMEM((CHUNK,), jnp.int32),
        pltpu.VMEM((CHUNK,), jnp.int32),
        pltpu.VMEM((CHUNK, F), jnp.float32),
        pltpu.VMEM_SHARED((NP, F), jnp.float32),
        pltpu.SemaphoreType.DMA,
    ],
)


def _dis_from(degp_ref):
    deg = degp_ref[0] + degp_ref[1] + 1.0
    return lax.rsqrt(jnp.maximum(deg, 1e-12))


def _tc_in(x_ref, w_ref, b_ref, o_ref):
    o_ref[...] = (jnp.dot(x_ref[...], w_ref[...],
                          preferred_element_type=jnp.float32) + b_ref[...])


def _tc_pre(degp_ref, h_ref, w_ref, o_ref):
    dis = _dis_from(degp_ref)
    hw = jnp.dot(h_ref[...], w_ref[...], preferred_element_type=jnp.float32)
    o_ref[...] = hw * dis


def _tc_mid(degp_ref, p_ref, g_ref, w_ref, bprev_ref, o_ref):
    dis = _dis_from(degp_ref)
    acc = p_ref[0] + p_ref[1] + g_ref[...]
    h = jnp.maximum(acc * dis + bprev_ref[...], 0.0)
    hw = jnp.dot(h, w_ref[...], preferred_element_type=jnp.float32)
    o_ref[...] = hw * dis


def _tc_out(degp_ref, p_ref, g_ref, w_ref, bprev_ref, b_ref, o_ref):
    dis = _dis_from(degp_ref)
    acc = p_ref[0] + p_ref[1] + g_ref[...]
    h = jnp.maximum(acc * dis + bprev_ref[...], 0.0)
    out = jnp.dot(h, w_ref[...], preferred_element_type=jnp.float32) + b_ref[...]
    o_ref[...] = out[:N]


_f32 = jnp.float32
_tc_in_call = pl.pallas_call(
    _tc_in, out_shape=jax.ShapeDtypeStruct((NP, F), _f32))
_tc_pre_call = pl.pallas_call(
    _tc_pre, out_shape=jax.ShapeDtypeStruct((NP, F), _f32))
_tc_mid_call = pl.pallas_call(
    _tc_mid, out_shape=jax.ShapeDtypeStruct((NP, F), _f32))
_tc_out_call = pl.pallas_call(
    _tc_out, out_shape=jax.ShapeDtypeStruct((N, F), _f32))


def kernel(x, edge_index, W_in, b_in, W0, b0, W1, b1, W_out, b_out):
    src = edge_index[0]
    dst = edge_index[1]
    pad_e = EP - E
    src_p = jnp.concatenate([src, jnp.zeros((pad_e,), jnp.int32)])
    dst_p = jnp.concatenate([dst, jnp.full((pad_e,), N, jnp.int32)])
    x_p = jnp.pad(x, ((0, NP - N), (0, 0)))
    z128 = jnp.zeros((NP, F), _f32)
    ones128 = jnp.ones((CHUNK, F), _f32)

    degp = _deg_call(dst_p, ones128, z128)
    h = _tc_in_call(x_p, W_in, b_in.reshape(1, F))
    g0 = _tc_pre_call(degp, h, W0)
    p0 = _conv_call(g0, src_p, dst_p, z128)
    g1 = _tc_mid_call(degp, p0, g0, W1, b0.reshape(1, F))
    p1 = _conv_call(g1, src_p, dst_p, z128)
    return _tc_out_call(degp, p1, g1, W_out,
                        b1.reshape(1, F), b_out.reshape(1, F))
